# baseline (speedup 1.0000x reference)
import functools

import jax
import jax.numpy as jnp
from jax import lax
from jax.experimental import pallas as pl
from jax.experimental.pallas import tpu as pltpu
from jax.experimental.pallas import tpu_sc as plsc

V = 1000000
D = 64
B = 16384
K = 20

NC = 2
NS = 16
L = 16
NW = NC * NS
BPW = B // NW
C = 32
NCHUNK = BPW // C
G = C // L
NEG_STREAMS = (C * K) // 128


VP = 500224


def _tc_pack_body(wt_ref, out_ref):
    b = wt_ref[...].T
    out_ref[:, 0:D] = b[0:256]
    out_ref[:, D:2 * D] = b[256:512]


_tc_pack = pl.pallas_call(
    _tc_pack_body,
    grid=(1954,),
    in_specs=[pl.BlockSpec((64, 512), lambda i: (0, i))],
    out_specs=pl.BlockSpec((256, 128), lambda i: (i, 0)),
    out_shape=jax.ShapeDtypeStruct((VP, 128), jnp.float32),
)


def _sc_scores(center, context, neg_flat, W_in2, W_out2):
    mesh = plsc.VectorSubcoreMesh(
        core_axis_name="c", subcore_axis_name="s", num_cores=NC,
        num_subcores=NS)

    @functools.partial(
        pl.kernel,
        out_type=(
            jax.ShapeDtypeStruct((B,), jnp.float32),
            jax.ShapeDtypeStruct((B * K,), jnp.float32),
        ),
        mesh=mesh,
        scratch_types=[
            pltpu.VMEM((C,), jnp.int32),
            pltpu.VMEM((C,), jnp.int32),
            pltpu.VMEM((C * K,), jnp.int32),
            pltpu.VMEM((C,), jnp.int32),
            pltpu.VMEM((C,), jnp.int32),
            pltpu.VMEM((C * K,), jnp.int32),
            pltpu.VMEM((C, 2 * D), jnp.float32),
            pltpu.VMEM((C, 2 * D), jnp.float32),
            pltpu.VMEM((C * K, 2 * D), jnp.float32),
            pltpu.VMEM((C,), jnp.float32),
            pltpu.VMEM((C * K,), jnp.float32),
            pltpu.SemaphoreType.DMA,
        ],
        compiler_params=pltpu.CompilerParams(
            needs_layout_passes=False, use_tc_tiling_on_sc=True),
    )
    def body(cen_hbm, ctx_hbm, neg_hbm, win_hbm, wout_hbm,
             pos_out, negs_out,
             cen_idx, ctx_idx, neg_idx, cen_pk, ctx_pk, neg_pk,
             cen_rows, ctx_rows, neg_rows, pos_buf, neg_buf, sem):
        wid = lax.axis_index("s") * NC + lax.axis_index("c")

        def chunk_body(ci, _):
            base = pl.multiple_of(wid * BPW + ci * C, C)
            nbase = pl.multiple_of(base * K, C * K)
            pltpu.sync_copy(cen_hbm.at[pl.ds(base, C)], cen_idx)
            pltpu.sync_copy(ctx_hbm.at[pl.ds(base, C)], ctx_idx)
            pltpu.sync_copy(neg_hbm.at[pl.ds(nbase, C * K)], neg_idx)
            def pk(v):
                return lax.shift_left(
                    lax.shift_right_logical(v, 9), 8) + (v & 255)

            for i in range(C // L):
                sl = pl.ds(i * L, L)
                cen_pk[sl] = pk(cen_idx[sl])
                ctx_pk[sl] = pk(ctx_idx[sl])
            for i in range(C * K // L):
                sl = pl.ds(i * L, L)
                neg_pk[sl] = pk(neg_idx[sl])
            copies = [
                pltpu.async_copy(win_hbm.at[cen_pk], cen_rows, sem),
                pltpu.async_copy(wout_hbm.at[ctx_pk], ctx_rows, sem),
            ]
            for j in range(NEG_STREAMS):
                copies.append(pltpu.async_copy(
                    wout_hbm.at[neg_pk.at[pl.ds(j * 128, 128)]],
                    neg_rows.at[pl.ds(j * 128, 128)], sem))
            for cp in copies:
                cp.wait()

            for g in range(G):
                lane = lax.iota(jnp.int32, L)
                row16 = g * L + lane
                nrow = [row16 * K + k for k in range(K)]
                def half(v):
                    return (lax.shift_right_logical(v, 8) & 1) * D

                halfc = half(cen_idx[pl.ds(g * L, L)])
                halfx = half(ctx_idx[pl.ds(g * L, L)])
                halfn = [
                    half(plsc.load_gather(neg_idx, [nrow[k]]))
                    for k in range(K)]
                zero = jnp.zeros((L,), jnp.float32)

                def dot_step(d, carry):
                    pos = carry[0]
                    accs = list(carry[1:])
                    rot = (d + lane) & (D - 1)
                    c_d = plsc.load_gather(cen_rows, [row16, halfc + rot])
                    x_d = plsc.load_gather(ctx_rows, [row16, halfx + rot])
                    pos = pos + c_d * x_d
                    new = [accs[k] + c_d * plsc.load_gather(
                        neg_rows, [nrow[k], halfn[k] + rot])
                        for k in range(K)]
                    return (pos, *new)

                res = lax.fori_loop(0, D, dot_step,
                                    (zero,) * (K + 1), unroll=2)
                pos_buf[pl.ds(g * L, L)] = res[0]
                for k in range(K):
                    plsc.store_scatter(neg_buf, [nrow[k]], res[1 + k])

            pltpu.sync_copy(pos_buf, pos_out.at[pl.ds(base, C)])
            pltpu.sync_copy(neg_buf, negs_out.at[pl.ds(nbase, C * K)])
            return ()

        lax.fori_loop(0, NCHUNK, chunk_body, ())

    return body(center, context, neg_flat, W_in2, W_out2)


def _tc_loss_body(pos_ref, neg_ref, out_ref):
    p = pos_ref[...]
    n = neg_ref[...]
    s1 = jnp.sum(-jnp.log(jax.nn.sigmoid(p) + 1e-10))
    s2 = jnp.sum(-jnp.log(jax.nn.sigmoid(-n) + 1e-10))
    out_ref[...] = jnp.broadcast_to((s1 + s2) * (1.0 / B), (1, 1))


_tc_loss = pl.pallas_call(
    _tc_loss_body,
    out_shape=jax.ShapeDtypeStruct((1, 1), jnp.float32),
)


def kernel(center, context, negatives, W_in, W_out):
    center = center.astype(jnp.int32)
    context = context.astype(jnp.int32)
    neg_flat = negatives.astype(jnp.int32).reshape(-1)
    pos, negs = _sc_scores(center, context, neg_flat,
                           _tc_pack(W_in.T), _tc_pack(W_out.T))
    loss = _tc_loss(pos.reshape(128, 128), negs.reshape(B * K // 128, 128))
    return loss[0, 0]


# speedup vs baseline: 3.0970x; 3.0970x over previous
import functools

import jax
import jax.numpy as jnp
from jax import lax
from jax.experimental import pallas as pl
from jax.experimental.pallas import tpu as pltpu
from jax.experimental.pallas import tpu_sc as plsc

V = 1000000
D = 64
B = 16384
K = 20

NC = 2
NS = 16
L = 16
NW = NC * NS
BPW = B // NW
C = 32
NCHUNK = BPW // C
G = C // L
NEG_STREAMS = (C * K) // 128


PCOLS = 4096
PGRID = -(-V // PCOLS)
VP = PGRID * PCOLS // 2


def _tc_pack_body(wt_ref, out_ref):
    eye = jnp.eye(D, dtype=jnp.float32)
    b = lax.dot_general(wt_ref[...], eye, (((0,), (0,)), ((), ())),
                        preferred_element_type=jnp.float32)
    for s in range(PCOLS // 512):
        out_ref[s * 256:(s + 1) * 256, 0:D] = b[s * 512:s * 512 + 256]
        out_ref[s * 256:(s + 1) * 256, D:2 * D] = (
            b[s * 512 + 256:(s + 1) * 512])


_tc_pack = pl.pallas_call(
    _tc_pack_body,
    grid=(PGRID,),
    in_specs=[pl.BlockSpec((D, PCOLS), lambda i: (0, i))],
    out_specs=pl.BlockSpec((PCOLS // 2, 128), lambda i: (i, 0)),
    out_shape=jax.ShapeDtypeStruct((VP, 128), jnp.float32),
)


def _sc_scores(center, context, neg_flat, W_in2, W_out2):
    mesh = plsc.VectorSubcoreMesh(
        core_axis_name="c", subcore_axis_name="s", num_cores=NC,
        num_subcores=NS)

    @functools.partial(
        pl.kernel,
        out_type=(
            jax.ShapeDtypeStruct((B,), jnp.float32),
            jax.ShapeDtypeStruct((B * K,), jnp.float32),
        ),
        mesh=mesh,
        scratch_types=[
            pltpu.VMEM((C,), jnp.int32),
            pltpu.VMEM((C,), jnp.int32),
            pltpu.VMEM((C * K,), jnp.int32),
            pltpu.VMEM((C,), jnp.int32),
            pltpu.VMEM((C,), jnp.int32),
            pltpu.VMEM((C * K,), jnp.int32),
            pltpu.VMEM((C, 2 * D), jnp.float32),
            pltpu.VMEM((C, 2 * D), jnp.float32),
            pltpu.VMEM((C * K, 2 * D), jnp.float32),
            pltpu.VMEM((C,), jnp.float32),
            pltpu.VMEM((C * K,), jnp.float32),
            pltpu.SemaphoreType.DMA,
        ],
        compiler_params=pltpu.CompilerParams(
            needs_layout_passes=False, use_tc_tiling_on_sc=True),
    )
    def body(cen_hbm, ctx_hbm, neg_hbm, win_hbm, wout_hbm,
             pos_out, negs_out,
             cen_idx, ctx_idx, neg_idx, cen_pk, ctx_pk, neg_pk,
             cen_rows, ctx_rows, neg_rows, pos_buf, neg_buf, sem):
        wid = lax.axis_index("s") * NC + lax.axis_index("c")

        def chunk_body(ci, _):
            base = pl.multiple_of(wid * BPW + ci * C, C)
            nbase = pl.multiple_of(base * K, C * K)
            pltpu.sync_copy(cen_hbm.at[pl.ds(base, C)], cen_idx)
            pltpu.sync_copy(ctx_hbm.at[pl.ds(base, C)], ctx_idx)
            pltpu.sync_copy(neg_hbm.at[pl.ds(nbase, C * K)], neg_idx)
            def pk(v):
                return lax.shift_left(
                    lax.shift_right_logical(v, 9), 8) + (v & 255)

            for i in range(C // L):
                sl = pl.ds(i * L, L)
                cen_pk[sl] = pk(cen_idx[sl])
                ctx_pk[sl] = pk(ctx_idx[sl])
            for i in range(C * K // L):
                sl = pl.ds(i * L, L)
                neg_pk[sl] = pk(neg_idx[sl])
            copies = [
                pltpu.async_copy(win_hbm.at[cen_pk], cen_rows, sem),
                pltpu.async_copy(wout_hbm.at[ctx_pk], ctx_rows, sem),
            ]
            for j in range(NEG_STREAMS):
                copies.append(pltpu.async_copy(
                    wout_hbm.at[neg_pk.at[pl.ds(j * 128, 128)]],
                    neg_rows.at[pl.ds(j * 128, 128)], sem))
            for cp in copies:
                cp.wait()

            for g in range(G):
                lane = lax.iota(jnp.int32, L)
                row16 = g * L + lane
                nrow = [row16 * K + k for k in range(K)]
                def half(v):
                    return (lax.shift_right_logical(v, 8) & 1) * D

                halfc = half(cen_idx[pl.ds(g * L, L)])
                halfx = half(ctx_idx[pl.ds(g * L, L)])
                halfn = [
                    half(plsc.load_gather(neg_idx, [nrow[k]]))
                    for k in range(K)]
                zero = jnp.zeros((L,), jnp.float32)

                def dot_step(d, carry):
                    pos = carry[0]
                    accs = list(carry[1:])
                    rot = (d + lane) & (D - 1)
                    c_d = plsc.load_gather(cen_rows, [row16, halfc + rot])
                    x_d = plsc.load_gather(ctx_rows, [row16, halfx + rot])
                    pos = pos + c_d * x_d
                    new = [accs[k] + c_d * plsc.load_gather(
                        neg_rows, [nrow[k], halfn[k] + rot])
                        for k in range(K)]
                    return (pos, *new)

                res = lax.fori_loop(0, D, dot_step,
                                    (zero,) * (K + 1), unroll=2)
                pos_buf[pl.ds(g * L, L)] = res[0]
                for k in range(K):
                    plsc.store_scatter(neg_buf, [nrow[k]], res[1 + k])

            pltpu.sync_copy(pos_buf, pos_out.at[pl.ds(base, C)])
            pltpu.sync_copy(neg_buf, negs_out.at[pl.ds(nbase, C * K)])
            return ()

        lax.fori_loop(0, NCHUNK, chunk_body, ())

    return body(center, context, neg_flat, W_in2, W_out2)


def _tc_loss_body(pos_ref, neg_ref, out_ref):
    p = pos_ref[...]
    n = neg_ref[...]
    s1 = jnp.sum(-jnp.log(jax.nn.sigmoid(p) + 1e-10))
    s2 = jnp.sum(-jnp.log(jax.nn.sigmoid(-n) + 1e-10))
    out_ref[...] = jnp.broadcast_to((s1 + s2) * (1.0 / B), (1, 1))


_tc_loss = pl.pallas_call(
    _tc_loss_body,
    out_shape=jax.ShapeDtypeStruct((1, 1), jnp.float32),
)


def kernel(center, context, negatives, W_in, W_out):
    center = center.astype(jnp.int32)
    context = context.astype(jnp.int32)
    neg_flat = negatives.astype(jnp.int32).reshape(-1)
    pos, negs = _sc_scores(center, context, neg_flat,
                           _tc_pack(W_in.T), _tc_pack(W_out.T))
    loss = _tc_loss(pos.reshape(128, 128), negs.reshape(B * K // 128, 128))
    return loss[0, 0]


# speedup vs baseline: 3.2643x; 1.0540x over previous
import functools

import jax
import jax.numpy as jnp
from jax import lax
from jax.experimental import pallas as pl
from jax.experimental.pallas import tpu as pltpu
from jax.experimental.pallas import tpu_sc as plsc

V = 1000000
D = 64
B = 16384
K = 20

NC = 2
NS = 16
L = 16
NW = NC * NS
BPW = B // NW
C = 16
NCHUNK = BPW // C
G = C // L
NEG_SPLITS = [(o, min(128, C * K - o)) for o in range(0, C * K, 128)]

PCOLS = 4096
PGRID = -(-V // PCOLS)
VP = PGRID * PCOLS // 2


def _tc_pack_body(wt_ref, out_ref):
    eye = jnp.eye(D, dtype=jnp.float32)
    b = lax.dot_general(wt_ref[...], eye, (((0,), (0,)), ((), ())),
                        preferred_element_type=jnp.float32)
    for s in range(PCOLS // 512):
        out_ref[s * 256:(s + 1) * 256, 0:D] = b[s * 512:s * 512 + 256]
        out_ref[s * 256:(s + 1) * 256, D:2 * D] = (
            b[s * 512 + 256:(s + 1) * 512])


_tc_pack = pl.pallas_call(
    _tc_pack_body,
    grid=(PGRID,),
    in_specs=[pl.BlockSpec((D, PCOLS), lambda i: (0, i))],
    out_specs=pl.BlockSpec((PCOLS // 2, 128), lambda i: (i, 0)),
    out_shape=jax.ShapeDtypeStruct((VP, 128), jnp.float32),
)


def _sc_scores(center, context, neg_flat, W_in2, W_out2):
    mesh = plsc.VectorSubcoreMesh(
        core_axis_name="c", subcore_axis_name="s", num_cores=NC,
        num_subcores=NS)

    def buf_set():
        return [
            pltpu.VMEM((C,), jnp.int32),
            pltpu.VMEM((C,), jnp.int32),
            pltpu.VMEM((C * K,), jnp.int32),
            pltpu.VMEM((C,), jnp.int32),
            pltpu.VMEM((C,), jnp.int32),
            pltpu.VMEM((C * K,), jnp.int32),
            pltpu.VMEM((C, 2 * D), jnp.float32),
            pltpu.VMEM((C, 2 * D), jnp.float32),
            pltpu.VMEM((C * K, 2 * D), jnp.float32),
            pltpu.VMEM((C,), jnp.float32),
            pltpu.VMEM((C * K,), jnp.float32),
            pltpu.SemaphoreType.DMA,
        ]

    @functools.partial(
        pl.kernel,
        out_type=(
            jax.ShapeDtypeStruct((B,), jnp.float32),
            jax.ShapeDtypeStruct((B * K,), jnp.float32),
        ),
        mesh=mesh,
        scratch_types=buf_set() + buf_set(),
        compiler_params=pltpu.CompilerParams(
            needs_layout_passes=False, use_tc_tiling_on_sc=True),
    )
    def body(cen_hbm, ctx_hbm, neg_hbm, win_hbm, wout_hbm,
             pos_out, negs_out, *scratch):
        bufs = (scratch[:12], scratch[12:])
        wid = lax.axis_index("s") * NC + lax.axis_index("c")

        def pk(v):
            return lax.shift_left(
                lax.shift_right_logical(v, 9), 8) + (v & 255)

        def issue(ci, bset):
            (cen_idx, ctx_idx, neg_idx, cen_pk, ctx_pk, neg_pk,
             cen_rows, ctx_rows, neg_rows, _pos_buf, _neg_buf, sem) = bset
            base = pl.multiple_of(wid * BPW + ci * C, C)
            nbase = pl.multiple_of(base * K, C * K)
            pltpu.sync_copy(cen_hbm.at[pl.ds(base, C)], cen_idx)
            pltpu.sync_copy(ctx_hbm.at[pl.ds(base, C)], ctx_idx)
            pltpu.sync_copy(neg_hbm.at[pl.ds(nbase, C * K)], neg_idx)
            for i in range(C // L):
                sl = pl.ds(i * L, L)
                cen_pk[sl] = pk(cen_idx[sl])
                ctx_pk[sl] = pk(ctx_idx[sl])
            for i in range(C * K // L):
                sl = pl.ds(i * L, L)
                neg_pk[sl] = pk(neg_idx[sl])
            pltpu.async_copy(win_hbm.at[cen_pk], cen_rows, sem)
            pltpu.async_copy(wout_hbm.at[ctx_pk], ctx_rows, sem)
            for (o, n) in NEG_SPLITS:
                pltpu.async_copy(wout_hbm.at[neg_pk.at[pl.ds(o, n)]],
                                 neg_rows.at[pl.ds(o, n)], sem)

        def wait_gathers(bset):
            (_ci, _xi, _ni, _cp, _xp, _np,
             cen_rows, ctx_rows, neg_rows, _p, _n, sem) = bset
            pltpu.make_async_copy(win_hbm.at[pl.ds(0, C)],
                                  cen_rows, sem).wait()
            pltpu.make_async_copy(win_hbm.at[pl.ds(0, C)],
                                  ctx_rows, sem).wait()
            for (o, n) in NEG_SPLITS:
                pltpu.make_async_copy(win_hbm.at[pl.ds(0, n)],
                                      neg_rows.at[pl.ds(o, n)], sem).wait()

        def compute(ci, bset):
            (cen_idx, ctx_idx, neg_idx, _cp, _xp, _np,
             cen_rows, ctx_rows, neg_rows, pos_buf, neg_buf, _sem) = bset
            base = pl.multiple_of(wid * BPW + ci * C, C)
            nbase = pl.multiple_of(base * K, C * K)
            for g in range(G):
                lane = lax.iota(jnp.int32, L)
                row16 = g * L + lane
                nrow = [row16 * K + k for k in range(K)]

                def half(v):
                    return (lax.shift_right_logical(v, 8) & 1) * D

                halfc = half(cen_idx[pl.ds(g * L, L)])
                halfx = half(ctx_idx[pl.ds(g * L, L)])
                halfn = [
                    half(plsc.load_gather(neg_idx, [nrow[k]]))
                    for k in range(K)]
                zero = jnp.zeros((L,), jnp.float32)

                def dot_step(d, carry):
                    pos = carry[0]
                    accs = list(carry[1:])
                    rot = (d + lane) & (D - 1)
                    c_d = plsc.load_gather(cen_rows, [row16, halfc + rot])
                    x_d = plsc.load_gather(ctx_rows, [row16, halfx + rot])
                    pos = pos + c_d * x_d
                    new = [accs[k] + c_d * plsc.load_gather(
                        neg_rows, [nrow[k], halfn[k] + rot])
                        for k in range(K)]
                    return (pos, *new)

                res = lax.fori_loop(0, D, dot_step,
                                    (zero,) * (K + 1), unroll=2)
                pos_buf[pl.ds(g * L, L)] = res[0]
                for k in range(K):
                    plsc.store_scatter(neg_buf, [nrow[k]], res[1 + k])

            pltpu.sync_copy(pos_buf, pos_out.at[pl.ds(base, C)])
            pltpu.sync_copy(neg_buf, negs_out.at[pl.ds(nbase, C * K)])

        issue(0, bufs[0])
        issue(1, bufs[1])

        def pair_body(u, _):
            for b in (0, 1):
                ci = u * 2 + b
                wait_gathers(bufs[b])
                compute(ci, bufs[b])

                @pl.when(ci + 2 < NCHUNK)
                def _():
                    issue(ci + 2, bufs[b])
            return ()

        lax.fori_loop(0, NCHUNK // 2, pair_body, ())

    return body(center, context, neg_flat, W_in2, W_out2)


def _tc_loss_body(pos_ref, neg_ref, out_ref):
    p = pos_ref[...]
    n = neg_ref[...]
    s1 = jnp.sum(-jnp.log(jax.nn.sigmoid(p) + 1e-10))
    s2 = jnp.sum(-jnp.log(jax.nn.sigmoid(-n) + 1e-10))
    out_ref[...] = jnp.broadcast_to((s1 + s2) * (1.0 / B), (1, 1))


_tc_loss = pl.pallas_call(
    _tc_loss_body,
    out_shape=jax.ShapeDtypeStruct((1, 1), jnp.float32),
)


def kernel(center, context, negatives, W_in, W_out):
    center = center.astype(jnp.int32)
    context = context.astype(jnp.int32)
    neg_flat = negatives.astype(jnp.int32).reshape(-1)
    pos, negs = _sc_scores(center, context, neg_flat,
                           _tc_pack(W_in.T), _tc_pack(W_out.T))
    loss = _tc_loss(pos.reshape(128, 128), negs.reshape(B * K // 128, 128))
    return loss[0, 0]


# speedup vs baseline: 3.9536x; 1.2112x over previous
import functools

import jax
import jax.numpy as jnp
from jax import lax
from jax.experimental import pallas as pl
from jax.experimental.pallas import tpu as pltpu
from jax.experimental.pallas import tpu_sc as plsc

V = 1000000
D = 64
B = 16384
K = 20

NC = 2
NS = 16
L = 16
NW = NC * NS
BPW = B // NW
C = 16
NCHUNK = BPW // C
G = C // L
NEG_SPLITS = [(o, min(128, C * K - o)) for o in range(0, C * K, 128)]

PCOLS = 4096
PGRID = -(-V // PCOLS)
VP = PGRID * PCOLS // 2


def _tc_pack_body(wt_ref, out_ref):
    eye = jnp.eye(D, dtype=jnp.float32)
    b = lax.dot_general(wt_ref[...], eye, (((0,), (0,)), ((), ())),
                        preferred_element_type=jnp.float32)
    for s in range(PCOLS // 512):
        out_ref[s * 256:(s + 1) * 256, 0:D] = b[s * 512:s * 512 + 256]
        out_ref[s * 256:(s + 1) * 256, D:2 * D] = (
            b[s * 512 + 256:(s + 1) * 512])


_tc_pack = pl.pallas_call(
    _tc_pack_body,
    grid=(PGRID,),
    in_specs=[pl.BlockSpec((D, PCOLS), lambda i: (0, i))],
    out_specs=pl.BlockSpec((PCOLS // 2, 128), lambda i: (i, 0)),
    out_shape=jax.ShapeDtypeStruct((VP, 128), jnp.float32),
)


NJ = -(-V // 128)
NWORK_VALID = 3907
PER_TILE = -(-NWORK_VALID // NW)


def _sc_pack(Wt):
    mesh = plsc.VectorSubcoreMesh(
        core_axis_name="c", subcore_axis_name="s", num_cores=NC,
        num_subcores=NS)

    def pbufs():
        return [
            pltpu.VMEM((D, 128), jnp.float32),
            pltpu.VMEM((D, 128), jnp.float32),
            pltpu.VMEM((128, 128), jnp.float32),
            pltpu.SemaphoreType.DMA,
            pltpu.SemaphoreType.DMA,
        ]

    @functools.partial(
        pl.kernel,
        out_type=jax.ShapeDtypeStruct((VP, 128), jnp.float32),
        mesh=mesh,
        scratch_types=pbufs() + pbufs(),
        compiler_params=pltpu.CompilerParams(
            needs_layout_passes=False, use_tc_tiling_on_sc=True),
    )
    def body(wt_hbm, out_hbm, *scratch):
        bufs = (scratch[:5], scratch[5:])
        wid = lax.axis_index("s") * NC + lax.axis_index("c")
        lane = lax.iota(jnp.int32, L)

        def witem(t):
            return wid + NW * t

        def issue_reads(t, bset):
            in1, in2, _o, rsem, _w = bset
            w = witem(t)

            @pl.when(w < NWORK_VALID)
            def _():
                a = lax.shift_right_logical(w, 1)
                parity = w & 1
                j1 = a * 4 + parity
                j2 = jnp.minimum(j1 + 2, NJ - 1)
                for (jj, inbuf) in ((j1, in1), (j2, in2)):
                    jcol = pl.multiple_of(jj * 128, 128)
                    for h in range(D // 8):
                        pltpu.async_copy(
                            wt_hbm.at[pl.ds(h * 8, 8), pl.ds(jcol, 128)],
                            inbuf.at[pl.ds(h * 8, 8), :], rsem)

        def drain_reads(bset):
            in1, in2, _o, rsem, _w = bset
            for inbuf in (in1, in2):
                for h in range(D // 8):
                    pltpu.make_async_copy(
                        wt_hbm.at[pl.ds(0, 8), pl.ds(0, 128)],
                        inbuf.at[pl.ds(h * 8, 8), :], rsem).wait()

        def drain_write(bset):
            _1, _2, out_t, _r, wsem = bset
            pltpu.make_async_copy(out_t, out_hbm.at[pl.ds(0, 128), :],
                                  wsem).wait()

        def transpose_write(t, bset):
            in1, in2, out_t, _r, wsem = bset
            w = witem(t)
            a = lax.shift_right_logical(w, 1)
            parity = w & 1

            def d_step(d0, _):
                for b01, inbuf in ((0, in1), (1, in2)):
                    dd = (d0 + lane) & (D - 1)
                    for l0 in range(0, 128, L):
                        ll = l0 + lane
                        vals = plsc.load_gather(inbuf, [dd, ll])
                        plsc.store_scatter(out_t, [ll, dd + b01 * D], vals)
                return ()

            lax.fori_loop(0, D, d_step, ())
            rstart = pl.multiple_of(a * 256 + parity * 128, 128)
            pltpu.async_copy(out_t, out_hbm.at[pl.ds(rstart, 128), :], wsem)

        issue_reads(0, bufs[0])
        issue_reads(1, bufs[1])

        def pair_body(u, _):
            for b in (0, 1):
                t = u * 2 + b
                w = witem(t)

                @pl.when(w < NWORK_VALID)
                def _():
                    drain_reads(bufs[b])

                    @pl.when(t >= 2)
                    def _():
                        drain_write(bufs[b])

                    transpose_write(t, bufs[b])
                issue_reads(t + 2, bufs[b])
            return ()

        lax.fori_loop(0, -(-PER_TILE // 2), pair_body, ())
        for b in (0, 1):
            drain_write(bufs[b])

    return body(Wt)


def _sc_scores(center, context, neg_flat, W_in2, W_out2):
    mesh = plsc.VectorSubcoreMesh(
        core_axis_name="c", subcore_axis_name="s", num_cores=NC,
        num_subcores=NS)

    def buf_set():
        return [
            pltpu.VMEM((C,), jnp.int32),
            pltpu.VMEM((C,), jnp.int32),
            pltpu.VMEM((C * K,), jnp.int32),
            pltpu.VMEM((C,), jnp.int32),
            pltpu.VMEM((C,), jnp.int32),
            pltpu.VMEM((C * K,), jnp.int32),
            pltpu.VMEM((C, 2 * D), jnp.float32),
            pltpu.VMEM((C, 2 * D), jnp.float32),
            pltpu.VMEM((C * K, 2 * D), jnp.float32),
            pltpu.VMEM((C,), jnp.float32),
            pltpu.VMEM((C * K,), jnp.float32),
            pltpu.SemaphoreType.DMA,
        ]

    @functools.partial(
        pl.kernel,
        out_type=(
            jax.ShapeDtypeStruct((B,), jnp.float32),
            jax.ShapeDtypeStruct((B * K,), jnp.float32),
        ),
        mesh=mesh,
        scratch_types=buf_set() + buf_set(),
        compiler_params=pltpu.CompilerParams(
            needs_layout_passes=False, use_tc_tiling_on_sc=True),
    )
    def body(cen_hbm, ctx_hbm, neg_hbm, win_hbm, wout_hbm,
             pos_out, negs_out, *scratch):
        bufs = (scratch[:12], scratch[12:])
        wid = lax.axis_index("s") * NC + lax.axis_index("c")

        def pk(v):
            return lax.shift_left(
                lax.shift_right_logical(v, 9), 8) + (v & 255)

        def issue(ci, bset):
            (cen_idx, ctx_idx, neg_idx, cen_pk, ctx_pk, neg_pk,
             cen_rows, ctx_rows, neg_rows, _pos_buf, _neg_buf, sem) = bset
            base = pl.multiple_of(wid * BPW + ci * C, C)
            nbase = pl.multiple_of(base * K, C * K)
            pltpu.sync_copy(cen_hbm.at[pl.ds(base, C)], cen_idx)
            pltpu.sync_copy(ctx_hbm.at[pl.ds(base, C)], ctx_idx)
            pltpu.sync_copy(neg_hbm.at[pl.ds(nbase, C * K)], neg_idx)
            for i in range(C // L):
                sl = pl.ds(i * L, L)
                cen_pk[sl] = pk(cen_idx[sl])
                ctx_pk[sl] = pk(ctx_idx[sl])
            for i in range(C * K // L):
                sl = pl.ds(i * L, L)
                neg_pk[sl] = pk(neg_idx[sl])
            pltpu.async_copy(win_hbm.at[cen_pk], cen_rows, sem)
            pltpu.async_copy(wout_hbm.at[ctx_pk], ctx_rows, sem)
            for (o, n) in NEG_SPLITS:
                pltpu.async_copy(wout_hbm.at[neg_pk.at[pl.ds(o, n)]],
                                 neg_rows.at[pl.ds(o, n)], sem)

        def wait_gathers(bset):
            (_ci, _xi, _ni, _cp, _xp, _np,
             cen_rows, ctx_rows, neg_rows, _p, _n, sem) = bset
            pltpu.make_async_copy(win_hbm.at[pl.ds(0, C)],
                                  cen_rows, sem).wait()
            pltpu.make_async_copy(win_hbm.at[pl.ds(0, C)],
                                  ctx_rows, sem).wait()
            for (o, n) in NEG_SPLITS:
                pltpu.make_async_copy(win_hbm.at[pl.ds(0, n)],
                                      neg_rows.at[pl.ds(o, n)], sem).wait()

        def compute(ci, bset):
            (cen_idx, ctx_idx, neg_idx, _cp, _xp, _np,
             cen_rows, ctx_rows, neg_rows, pos_buf, neg_buf, _sem) = bset
            base = pl.multiple_of(wid * BPW + ci * C, C)
            nbase = pl.multiple_of(base * K, C * K)
            for g in range(G):
                lane = lax.iota(jnp.int32, L)
                row16 = g * L + lane
                nrow = [row16 * K + k for k in range(K)]

                def half(v):
                    return (lax.shift_right_logical(v, 8) & 1) * D

                halfc = half(cen_idx[pl.ds(g * L, L)])
                halfx = half(ctx_idx[pl.ds(g * L, L)])
                halfn = [
                    half(plsc.load_gather(neg_idx, [nrow[k]]))
                    for k in range(K)]
                zero = jnp.zeros((L,), jnp.float32)

                def dot_step(d, carry):
                    pos = carry[0]
                    accs = list(carry[1:])
                    rot = (d + lane) & (D - 1)
                    c_d = plsc.load_gather(cen_rows, [row16, halfc + rot])
                    x_d = plsc.load_gather(ctx_rows, [row16, halfx + rot])
                    pos = pos + c_d * x_d
                    new = [accs[k] + c_d * plsc.load_gather(
                        neg_rows, [nrow[k], halfn[k] + rot])
                        for k in range(K)]
                    return (pos, *new)

                res = lax.fori_loop(0, D, dot_step,
                                    (zero,) * (K + 1), unroll=2)
                pos_buf[pl.ds(g * L, L)] = res[0]
                for k in range(K):
                    plsc.store_scatter(neg_buf, [nrow[k]], res[1 + k])

            pltpu.sync_copy(pos_buf, pos_out.at[pl.ds(base, C)])
            pltpu.sync_copy(neg_buf, negs_out.at[pl.ds(nbase, C * K)])

        issue(0, bufs[0])
        issue(1, bufs[1])

        def pair_body(u, _):
            for b in (0, 1):
                ci = u * 2 + b
                wait_gathers(bufs[b])
                compute(ci, bufs[b])

                @pl.when(ci + 2 < NCHUNK)
                def _():
                    issue(ci + 2, bufs[b])
            return ()

        lax.fori_loop(0, NCHUNK // 2, pair_body, ())

    return body(center, context, neg_flat, W_in2, W_out2)


def _tc_loss_body(pos_ref, neg_ref, out_ref):
    p = pos_ref[...]
    n = neg_ref[...]
    s1 = jnp.sum(-jnp.log(jax.nn.sigmoid(p) + 1e-10))
    s2 = jnp.sum(-jnp.log(jax.nn.sigmoid(-n) + 1e-10))
    out_ref[...] = jnp.broadcast_to((s1 + s2) * (1.0 / B), (1, 1))


_tc_loss = pl.pallas_call(
    _tc_loss_body,
    out_shape=jax.ShapeDtypeStruct((1, 1), jnp.float32),
)


def kernel(center, context, negatives, W_in, W_out):
    center = center.astype(jnp.int32)
    context = context.astype(jnp.int32)
    neg_flat = negatives.astype(jnp.int32).reshape(-1)
    pos, negs = _sc_scores(center, context, neg_flat,
                           _sc_pack(W_in.T), _tc_pack(W_out.T))
    loss = _tc_loss(pos.reshape(128, 128), negs.reshape(B * K // 128, 128))
    return loss[0, 0]
